# baseline (speedup 1.0000x reference)
import functools

import jax
import jax.numpy as jnp
from jax import lax
from jax.experimental import pallas as pl
from jax.experimental.pallas import tpu as pltpu
from jax.experimental.pallas import tpu_sc as plsc

N_VOCAB = 1000000
EMBED_DIM = 32
BATCH = 16384
N_FIELDS = 26

NC = 2
NS = 16
NW = NC * NS
L = 16

S_PER_W = BATCH // NW
CHUNK = 8
N_CHUNKS = S_PER_W // CHUNK
IDX_PER_CHUNK = CHUNK * N_FIELDS
IDX_PER_W = S_PER_W * N_FIELDS
SUB_N = IDX_PER_CHUNK
SUBS = IDX_PER_CHUNK // SUB_N


def _fm_body(x_hbm, xo_hbm, w0_hbm, bsum_hbm, emb_hbm, out_hbm,
             xidx, xoff, embv, bsv, logits, w0v, sem):
    wid = lax.axis_index("s") * NC + lax.axis_index("c")
    base_idx = pl.multiple_of(wid * IDX_PER_W, IDX_PER_W)

    pltpu.sync_copy(x_hbm.at[pl.ds(base_idx, IDX_PER_W)], xidx)
    pltpu.sync_copy(xo_hbm.at[pl.ds(base_idx, IDX_PER_W)], xoff.at[pl.ds(0, IDX_PER_W)])
    pltpu.sync_copy(w0_hbm, w0v)

    lane = jnp.arange(L, dtype=jnp.int32)

    def _streams(c, b):
        coff = pl.multiple_of(c * IDX_PER_CHUNK, IDX_PER_CHUNK)
        ops = []
        for j in range(SUBS):
            n = SUB_N
            o = j * SUB_N
            idx_ref = xidx.at[pl.ds(coff + o, n)]
            ops.append((emb_hbm.at[idx_ref], embv.at[b, pl.ds(o, n)]))
        return ops

    def _issue(c, b):
        for src, dst in _streams(c, b):
            pltpu.async_copy(src, dst, sem)

    def _drain(c, b):
        for src, dst in _streams(c, b):
            pltpu.make_async_copy(src, dst, sem).wait()

    _issue(0, 0)
    out_base = pl.multiple_of(wid * S_PER_W, S_PER_W)
    pltpu.sync_copy(bsum_hbm.at[pl.ds(out_base, S_PER_W)], bsv)
    w0vec = w0v[...]

    @pl.loop(0, N_CHUNKS)
    def _chunk(c):
        b = lax.rem(c, 2)
        _drain(c, b)

        @pl.when(c + 1 < N_CHUNKS)
        def _():
            _issue(c + 1, 1 - b)

        coff = pl.multiple_of(c * IDX_PER_CHUNK, IDX_PER_CHUNK)

        @pl.loop(0, CHUNK)
        def _sample(i):
            kb = i * N_FIELDS
            s0 = jnp.zeros((L,), jnp.float32)
            s1 = jnp.zeros((L,), jnp.float32)
            q0 = jnp.zeros((L,), jnp.float32)
            q1 = jnp.zeros((L,), jnp.float32)
            gbase = pl.multiple_of(coff + kb, 2)
            ov0 = xoff[pl.ds(gbase, L)]
            ov1 = xoff[pl.ds(gbase + L, L)]
            for f in range(N_FIELDS):
                o = ov0[f] if f < L else ov1[f - L]
                r0 = embv[b, kb + f, pl.ds(o, L)]
                r1 = embv[b, kb + f, pl.ds(o + L, L)]
                s0 = s0 + r0
                q0 = q0 + r0 * r0
                s1 = s1 + r1
                q1 = q1 + r1 * r1
            u = (s0 * s0 - q0) + (s1 * s1 - q1)
            r = jnp.sum(u)
            plsc.store_scatter(
                logits,
                [jnp.broadcast_to(c * CHUNK + i, (L,)).astype(jnp.int32)],
                jnp.broadcast_to(r, (L,)).astype(jnp.float32),
                mask=lane == 0)

    @pl.loop(0, S_PER_W // L)
    def _fin(g):
        off = pl.multiple_of(g * L, L)
        z = 0.5 * logits[pl.ds(off, L)] + bsv[pl.ds(off, L)] + w0vec
        logits[pl.ds(off, L)] = 5.5 / (1.0 + jnp.exp(-z))

    pltpu.sync_copy(logits, out_hbm.at[pl.ds(out_base, S_PER_W)])


NJ_FULL = N_VOCAB // 128
TAIL_V = N_VOCAB - NJ_FULL * 128
BLK_PER_W = NJ_FULL // NW
NJ_LEFT = NJ_FULL - BLK_PER_W * NW


def _tr_body(t2_hbm, out_hbm, stage, outb, tstage, sem_in, sem_out):
    wid = lax.axis_index("s") * NC + lax.axis_index("c")
    lane = jnp.arange(L, dtype=jnp.int32)

    def _issue_in(j, b):
        cps = []
        for i in range(4):
            cps.append(pltpu.async_copy(
                t2_hbm.at[pl.ds(i * 8, 8), pl.ds(j * 128, 128)],
                stage.at[b, pl.ds(i * 8, 8), :], sem_in))
        return cps

    def _drain_in(j, b):
        for i in range(4):
            pltpu.make_async_copy(
                t2_hbm.at[pl.ds(i * 8, 8), pl.ds(j * 128, 128)],
                stage.at[b, pl.ds(i * 8, 8), :], sem_in).wait()

    def _out_descr(j, b):
        return (outb.at[b], out_hbm.at[pl.ds(j * 32, 32), :])

    def _transpose(b):
        for k in range(L):
            perm = (lane + k) & (L - 1)
            rows0 = perm >> 2
            colb = (perm & 3) << 5
            for dg in range(2):
                drow = dg * L + lane
                cols = colb + drow
                for cg in range(8):
                    g = plsc.load_gather(stage.at[b], [drow, cg * L + perm])
                    plsc.store_scatter(outb.at[b], [rows0 + cg * 4, cols], g)

    base = pl.multiple_of(wid * BLK_PER_W, 4)

    _issue_in(base, 0)
    _issue_in(base + 1, 1)

    @pl.loop(0, BLK_PER_W)
    def _blk(jj):
        j = base + jj
        b = lax.rem(jj, 2)
        _drain_in(j, b)

        @pl.when(jj >= 2)
        def _():
            src_o, dst_o = _out_descr(j - 2, b)
            pltpu.make_async_copy(src_o, dst_o, sem_out).wait()

        _transpose(b)

        @pl.when(jj + 2 < BLK_PER_W)
        def _():
            _issue_in(j + 2, b)

        src_o, dst_o = _out_descr(j, b)
        pltpu.async_copy(src_o, dst_o, sem_out)

    for jj in (BLK_PER_W - 2, BLK_PER_W - 1):
        b = jj % 2
        src_o, dst_o = _out_descr(base + jj, b)
        pltpu.make_async_copy(src_o, dst_o, sem_out).wait()

    @pl.when(wid < NJ_LEFT)
    def _():
        j = NW * BLK_PER_W + wid
        for cp in _issue_in(j, 0):
            pass
        _drain_in(j, 0)
        _transpose(0)
        src_o, dst_o = _out_descr(j, 0)
        pltpu.async_copy(src_o, dst_o, sem_out).wait()

    @pl.when(wid == NW - 1)
    def _():
        cps = []
        for d in range(32):
            cps.append(pltpu.async_copy(
                t2_hbm.at[pl.ds(d, 1), pl.ds(NJ_FULL * 128, TAIL_V)],
                tstage.at[pl.ds(d, 1), :], sem_in))
        for cp in cps:
            cp.wait()
        for k in range(L):
            perm = (lane + k) & (L - 1)
            rows0 = perm >> 2
            colb = (perm & 3) << 5
            for dg in range(2):
                drow = dg * L + lane
                cols = colb + drow
                for cg in range(4):
                    g = plsc.load_gather(tstage, [drow, cg * L + perm])
                    plsc.store_scatter(
                        outb.at[0, pl.ds(0, 16)], [rows0 + cg * 4, cols], g)
        pltpu.async_copy(
            outb.at[0, pl.ds(0, 16)],
            out_hbm.at[pl.ds(NJ_FULL * 32, 16), :], sem_out).wait()


_tr_call = pl.kernel(
    _tr_body,
    out_type=jax.ShapeDtypeStruct((N_VOCAB // 4, 128), jnp.float32),
    mesh=plsc.VectorSubcoreMesh(core_axis_name="c", subcore_axis_name="s"),
    scratch_types=[
        pltpu.VMEM((2, 32, 128), jnp.float32),
        pltpu.VMEM((2, 32, 128), jnp.float32),
        pltpu.VMEM((32, TAIL_V), jnp.float32),
        pltpu.SemaphoreType.DMA,
        pltpu.SemaphoreType.DMA,
    ],
    compiler_params=pltpu.CompilerParams(
        needs_layout_passes=False, use_tc_tiling_on_sc=True),
)


def _bias_body(x_hbm, bias_hbm, out_hbm, xidx, bv, ov, sem):
    wid = lax.axis_index("s") * NC + lax.axis_index("c")
    base_idx = pl.multiple_of(wid * IDX_PER_W, IDX_PER_W)
    pltpu.sync_copy(x_hbm.at[pl.ds(base_idx, IDX_PER_W)], xidx)
    lane = jnp.arange(L, dtype=jnp.int32)

    cps = []
    for j in range(8):
        n = IDX_PER_W // 8
        idx_ref = xidx.at[pl.ds(j * n, n)]
        cps.append(pltpu.async_copy(bias_hbm.at[idx_ref], bv.at[pl.ds(j * n, n)], sem))
    for cp in cps:
        cp.wait()

    @pl.loop(0, S_PER_W // L)
    def _grp(g):
        brow = (g * L + lane) * N_FIELDS
        bsum = jnp.zeros((L,), jnp.float32)
        for f in range(N_FIELDS):
            bsum = bsum + plsc.load_gather(bv, [brow + f])
        off = pl.multiple_of(g * L, L)
        ov[pl.ds(off, L)] = bsum

    out_base = pl.multiple_of(wid * S_PER_W, S_PER_W)
    pltpu.sync_copy(ov, out_hbm.at[pl.ds(out_base, S_PER_W)])


_bias_call = pl.kernel(
    _bias_body,
    out_type=jax.ShapeDtypeStruct((BATCH,), jnp.float32),
    mesh=plsc.VectorSubcoreMesh(core_axis_name="c", subcore_axis_name="s"),
    scratch_types=[
        pltpu.VMEM((IDX_PER_W,), jnp.int32),
        pltpu.VMEM((IDX_PER_W,), jnp.float32),
        pltpu.VMEM((S_PER_W,), jnp.float32),
        pltpu.SemaphoreType.DMA,
    ],
    compiler_params=pltpu.CompilerParams(
        needs_layout_passes=False, use_tc_tiling_on_sc=False),
)


_fm_call = pl.kernel(
    _fm_body,
    out_type=jax.ShapeDtypeStruct((BATCH,), jnp.float32),
    mesh=plsc.VectorSubcoreMesh(core_axis_name="c", subcore_axis_name="s"),
    scratch_types=[
        pltpu.VMEM((IDX_PER_W,), jnp.int32),
        pltpu.VMEM((IDX_PER_W + 2 * L,), jnp.int32),
        pltpu.VMEM((2, IDX_PER_CHUNK, 128), jnp.float32),
        pltpu.VMEM((S_PER_W,), jnp.float32),
        pltpu.VMEM((S_PER_W,), jnp.float32),
        pltpu.VMEM((L,), jnp.float32),
        pltpu.SemaphoreType.DMA,
    ],
    compiler_params=pltpu.CompilerParams(
        needs_layout_passes=False, use_tc_tiling_on_sc=True),
)


def kernel(X, w0, bias_table, emb_table):
    x_flat = X.reshape(-1).astype(jnp.int32)
    x4 = x_flat // 4
    xo = (x_flat % 4) * EMBED_DIM
    bias_flat = bias_table.reshape(-1)
    emb128 = _tr_call(emb_table.T)
    w0v = jnp.broadcast_to(w0.astype(jnp.float32), (L,))
    bias_sums = _bias_call(x_flat, bias_flat)
    return _fm_call(x4, xo, w0v, bias_sums, emb128)


# speedup vs baseline: 1.4516x; 1.4516x over previous
import functools

import jax
import jax.numpy as jnp
from jax import lax
from jax.experimental import pallas as pl
from jax.experimental.pallas import tpu as pltpu
from jax.experimental.pallas import tpu_sc as plsc

N_VOCAB = 1000000
EMBED_DIM = 32
BATCH = 16384
N_FIELDS = 26

NC = 2
NS = 16
NW = NC * NS
L = 16

S_PER_W = BATCH // NW
CHUNK = 8
N_CHUNKS = S_PER_W // CHUNK
IDX_PER_CHUNK = CHUNK * N_FIELDS
IDX_PER_W = S_PER_W * N_FIELDS
SUB_N = IDX_PER_CHUNK
SUBS = IDX_PER_CHUNK // SUB_N


def _fm_body(x_hbm, xo_hbm, w0_hbm, bsum_hbm, emb_hbm, out_hbm,
             xidx, xoff, embv, bsv, logits, w0v, sem):
    wid = lax.axis_index("s") * NC + lax.axis_index("c")
    base_idx = pl.multiple_of(wid * IDX_PER_W, IDX_PER_W)

    pltpu.sync_copy(x_hbm.at[pl.ds(base_idx, IDX_PER_W)], xidx)
    pltpu.sync_copy(xo_hbm.at[pl.ds(base_idx, IDX_PER_W)], xoff.at[pl.ds(0, IDX_PER_W)])
    pltpu.sync_copy(w0_hbm, w0v)

    lane = jnp.arange(L, dtype=jnp.int32)

    def _streams(c, b):
        coff = pl.multiple_of(c * IDX_PER_CHUNK, IDX_PER_CHUNK)
        ops = []
        for j in range(SUBS):
            n = SUB_N
            o = j * SUB_N
            idx_ref = xidx.at[pl.ds(coff + o, n)]
            ops.append((emb_hbm.at[idx_ref], embv.at[b, pl.ds(o, n)]))
        return ops

    def _issue(c, b):
        for src, dst in _streams(c, b):
            pltpu.async_copy(src, dst, sem)

    def _drain(c, b):
        for src, dst in _streams(c, b):
            pltpu.make_async_copy(src, dst, sem).wait()

    _issue(0, 0)
    out_base = pl.multiple_of(wid * S_PER_W, S_PER_W)
    pltpu.sync_copy(bsum_hbm.at[pl.ds(out_base, S_PER_W)], bsv)
    w0vec = w0v[...]

    @pl.loop(0, N_CHUNKS)
    def _chunk(c):
        b = lax.rem(c, 2)
        _drain(c, b)

        @pl.when(c + 1 < N_CHUNKS)
        def _():
            _issue(c + 1, 1 - b)

        coff = pl.multiple_of(c * IDX_PER_CHUNK, IDX_PER_CHUNK)

        @pl.loop(0, CHUNK)
        def _sample(i):
            kb = i * N_FIELDS
            s0 = jnp.zeros((L,), jnp.float32)
            s1 = jnp.zeros((L,), jnp.float32)
            q0 = jnp.zeros((L,), jnp.float32)
            q1 = jnp.zeros((L,), jnp.float32)
            gbase = pl.multiple_of(coff + kb, 2)
            ov0 = xoff[pl.ds(gbase, L)]
            ov1 = xoff[pl.ds(gbase + L, L)]
            for f in range(N_FIELDS):
                o = ov0[f] if f < L else ov1[f - L]
                r0 = embv[b, kb + f, pl.ds(o, L)]
                r1 = embv[b, kb + f, pl.ds(o + L, L)]
                s0 = s0 + r0
                q0 = q0 + r0 * r0
                s1 = s1 + r1
                q1 = q1 + r1 * r1
            u = (s0 * s0 - q0) + (s1 * s1 - q1)
            r = jnp.sum(u)
            plsc.store_scatter(
                logits,
                [jnp.broadcast_to(c * CHUNK + i, (L,)).astype(jnp.int32)],
                jnp.broadcast_to(r, (L,)).astype(jnp.float32),
                mask=lane == 0)

    @pl.loop(0, S_PER_W // L)
    def _fin(g):
        off = pl.multiple_of(g * L, L)
        z = 0.5 * logits[pl.ds(off, L)] + bsv[pl.ds(off, L)] + w0vec
        logits[pl.ds(off, L)] = 5.5 / (1.0 + jnp.exp(-z))

    pltpu.sync_copy(logits, out_hbm.at[pl.ds(out_base, S_PER_W)])


NJ_FULL = N_VOCAB // 128
TAIL_V = N_VOCAB - NJ_FULL * 128
BLK_PER_W = NJ_FULL // NW
NJ_LEFT = NJ_FULL - BLK_PER_W * NW


def _tr_body(t2_hbm, out_hbm, stage, outb, tstage, sem_in, sem_out):
    wid = lax.axis_index("s") * NC + lax.axis_index("c")
    lane = jnp.arange(L, dtype=jnp.int32)

    def _issue_in(j, b):
        cps = []
        for i in range(4):
            cps.append(pltpu.async_copy(
                t2_hbm.at[pl.ds(i * 8, 8), pl.ds(j * 128, 128)],
                stage.at[b, pl.ds(i * 8, 8), :], sem_in))
        return cps

    def _drain_in(j, b):
        for i in range(4):
            pltpu.make_async_copy(
                t2_hbm.at[pl.ds(i * 8, 8), pl.ds(j * 128, 128)],
                stage.at[b, pl.ds(i * 8, 8), :], sem_in).wait()

    def _out_descr(j, b):
        return (outb.at[b], out_hbm.at[pl.ds(j * 32, 32), :])

    def _transpose(b):
        for k in range(L):
            perm = (lane + k) & (L - 1)
            rows0 = perm >> 2
            colb = (perm & 3) << 5
            for dg in range(2):
                drow = dg * L + lane
                cols = colb + drow
                gs = [plsc.load_gather(stage.at[b], [drow, cg * L + perm])
                      for cg in range(8)]
                for cg in range(8):
                    plsc.store_scatter(outb.at[b], [rows0 + cg * 4, cols], gs[cg])

    base = pl.multiple_of(wid * BLK_PER_W, 4)

    _issue_in(base, 0)
    _issue_in(base + 1, 1)

    @pl.loop(0, BLK_PER_W)
    def _blk(jj):
        j = base + jj
        b = lax.rem(jj, 2)
        _drain_in(j, b)

        @pl.when(jj >= 2)
        def _():
            src_o, dst_o = _out_descr(j - 2, b)
            pltpu.make_async_copy(src_o, dst_o, sem_out).wait()

        _transpose(b)

        @pl.when(jj + 2 < BLK_PER_W)
        def _():
            _issue_in(j + 2, b)

        src_o, dst_o = _out_descr(j, b)
        pltpu.async_copy(src_o, dst_o, sem_out)

    for jj in (BLK_PER_W - 2, BLK_PER_W - 1):
        b = jj % 2
        src_o, dst_o = _out_descr(base + jj, b)
        pltpu.make_async_copy(src_o, dst_o, sem_out).wait()

    @pl.when(wid < NJ_LEFT)
    def _():
        j = NW * BLK_PER_W + wid
        for cp in _issue_in(j, 0):
            pass
        _drain_in(j, 0)
        _transpose(0)
        src_o, dst_o = _out_descr(j, 0)
        pltpu.async_copy(src_o, dst_o, sem_out).wait()

    @pl.when(wid == NW - 1)
    def _():
        cps = []
        for d in range(32):
            cps.append(pltpu.async_copy(
                t2_hbm.at[pl.ds(d, 1), pl.ds(NJ_FULL * 128, TAIL_V)],
                tstage.at[pl.ds(d, 1), :], sem_in))
        for cp in cps:
            cp.wait()
        for k in range(L):
            perm = (lane + k) & (L - 1)
            rows0 = perm >> 2
            colb = (perm & 3) << 5
            for dg in range(2):
                drow = dg * L + lane
                cols = colb + drow
                for cg in range(4):
                    g = plsc.load_gather(tstage, [drow, cg * L + perm])
                    plsc.store_scatter(
                        outb.at[0, pl.ds(0, 16)], [rows0 + cg * 4, cols], g)
        pltpu.async_copy(
            outb.at[0, pl.ds(0, 16)],
            out_hbm.at[pl.ds(NJ_FULL * 32, 16), :], sem_out).wait()


_tr_call = pl.kernel(
    _tr_body,
    out_type=jax.ShapeDtypeStruct((N_VOCAB // 4, 128), jnp.float32),
    mesh=plsc.VectorSubcoreMesh(core_axis_name="c", subcore_axis_name="s"),
    scratch_types=[
        pltpu.VMEM((2, 32, 128), jnp.float32),
        pltpu.VMEM((2, 32, 128), jnp.float32),
        pltpu.VMEM((32, TAIL_V), jnp.float32),
        pltpu.SemaphoreType.DMA,
        pltpu.SemaphoreType.DMA,
    ],
    compiler_params=pltpu.CompilerParams(
        needs_layout_passes=False, use_tc_tiling_on_sc=True),
)


def _bias_body(x_hbm, bias_hbm, out_hbm, xidx, bv, ov, sem):
    wid = lax.axis_index("s") * NC + lax.axis_index("c")
    base_idx = pl.multiple_of(wid * IDX_PER_W, IDX_PER_W)
    pltpu.sync_copy(x_hbm.at[pl.ds(base_idx, IDX_PER_W)], xidx)
    lane = jnp.arange(L, dtype=jnp.int32)

    cps = []
    for j in range(8):
        n = IDX_PER_W // 8
        idx_ref = xidx.at[pl.ds(j * n, n)]
        cps.append(pltpu.async_copy(bias_hbm.at[idx_ref], bv.at[pl.ds(j * n, n)], sem))
    for cp in cps:
        cp.wait()

    @pl.loop(0, S_PER_W // L)
    def _grp(g):
        brow = (g * L + lane) * N_FIELDS
        bsum = jnp.zeros((L,), jnp.float32)
        for f in range(N_FIELDS):
            bsum = bsum + plsc.load_gather(bv, [brow + f])
        off = pl.multiple_of(g * L, L)
        ov[pl.ds(off, L)] = bsum

    out_base = pl.multiple_of(wid * S_PER_W, S_PER_W)
    pltpu.sync_copy(ov, out_hbm.at[pl.ds(out_base, S_PER_W)])


_bias_call = pl.kernel(
    _bias_body,
    out_type=jax.ShapeDtypeStruct((BATCH,), jnp.float32),
    mesh=plsc.VectorSubcoreMesh(core_axis_name="c", subcore_axis_name="s"),
    scratch_types=[
        pltpu.VMEM((IDX_PER_W,), jnp.int32),
        pltpu.VMEM((IDX_PER_W,), jnp.float32),
        pltpu.VMEM((S_PER_W,), jnp.float32),
        pltpu.SemaphoreType.DMA,
    ],
    compiler_params=pltpu.CompilerParams(
        needs_layout_passes=False, use_tc_tiling_on_sc=False),
)


_fm_call = pl.kernel(
    _fm_body,
    out_type=jax.ShapeDtypeStruct((BATCH,), jnp.float32),
    mesh=plsc.VectorSubcoreMesh(core_axis_name="c", subcore_axis_name="s"),
    scratch_types=[
        pltpu.VMEM((IDX_PER_W,), jnp.int32),
        pltpu.VMEM((IDX_PER_W + 2 * L,), jnp.int32),
        pltpu.VMEM((2, IDX_PER_CHUNK, 128), jnp.float32),
        pltpu.VMEM((S_PER_W,), jnp.float32),
        pltpu.VMEM((S_PER_W,), jnp.float32),
        pltpu.VMEM((L,), jnp.float32),
        pltpu.SemaphoreType.DMA,
    ],
    compiler_params=pltpu.CompilerParams(
        needs_layout_passes=False, use_tc_tiling_on_sc=True),
)


def kernel(X, w0, bias_table, emb_table):
    x_flat = X.reshape(-1).astype(jnp.int32)
    x4 = x_flat // 4
    xo = (x_flat % 4) * EMBED_DIM
    bias_flat = bias_table.reshape(-1)
    emb128 = _tr_call(emb_table.T)
    w0v = jnp.broadcast_to(w0.astype(jnp.float32), (L,))
    bias_sums = _bias_call(x_flat, bias_flat)
    return _fm_call(x4, xo, w0v, bias_sums, emb128)


# speedup vs baseline: 1.4536x; 1.0014x over previous
import functools

import jax
import jax.numpy as jnp
from jax import lax
from jax.experimental import pallas as pl
from jax.experimental.pallas import tpu as pltpu
from jax.experimental.pallas import tpu_sc as plsc

N_VOCAB = 1000000
EMBED_DIM = 32
BATCH = 16384
N_FIELDS = 26

NC = 2
NS = 16
NW = NC * NS
L = 16

S_PER_W = BATCH // NW
CHUNK = 32
N_CHUNKS = S_PER_W // CHUNK
IDX_PER_CHUNK = CHUNK * N_FIELDS
IDX_PER_W = S_PER_W * N_FIELDS
SUB_N = IDX_PER_CHUNK
SUBS = IDX_PER_CHUNK // SUB_N


def _fm_body(x_hbm, w0_hbm, bias_hbm, emb_hbm, out_hbm,
             xidx, embv, biasv, logits, stage, w0v, sem):
    wid = lax.axis_index("s") * NC + lax.axis_index("c")
    base_idx = pl.multiple_of(wid * IDX_PER_W, IDX_PER_W)

    pltpu.sync_copy(x_hbm.at[pl.ds(base_idx, IDX_PER_W)], xidx)
    pltpu.sync_copy(w0_hbm, w0v)

    lane = jnp.arange(L, dtype=jnp.int32)

    def _streams(c, b):
        coff = pl.multiple_of(c * IDX_PER_CHUNK, IDX_PER_CHUNK)
        ops = []
        for j in range(SUBS):
            n = SUB_N
            o = j * SUB_N
            idx_ref = xidx.at[pl.ds(coff + o, n)]
            ops.append((emb_hbm.at[idx_ref], embv.at[b, pl.ds(o, n)]))
            ops.append((bias_hbm.at[idx_ref], biasv.at[b, pl.ds(o, n)]))
        return ops

    def _issue(c, b):
        for src, dst in _streams(c, b):
            pltpu.async_copy(src, dst, sem)

    def _drain(c, b):
        for src, dst in _streams(c, b):
            pltpu.make_async_copy(src, dst, sem).wait()

    _issue(0, 0)
    w0vec = w0v[...]

    @pl.loop(0, N_CHUNKS)
    def _chunk(c):
        b = lax.rem(c, 2)
        _drain(c, b)

        @pl.when(c + 1 < N_CHUNKS)
        def _():
            _issue(c + 1, 1 - b)

        @pl.loop(0, CHUNK)
        def _sample(i):
            kb = i * N_FIELDS
            s0 = jnp.zeros((L,), jnp.float32)
            s1 = jnp.zeros((L,), jnp.float32)
            q0 = jnp.zeros((L,), jnp.float32)
            q1 = jnp.zeros((L,), jnp.float32)
            for f in range(N_FIELDS):
                r0 = embv[b, kb + f, pl.ds(0, L)]
                r1 = embv[b, kb + f, pl.ds(L, L)]
                s0 = s0 + r0
                q0 = q0 + r0 * r0
                s1 = s1 + r1
                q1 = q1 + r1 * r1
            u = (s0 * s0 - q0) + (s1 * s1 - q1)
            stage[i, pl.ds(0, L)] = u

        for g in range(CHUNK // L):
            rows = g * L + lane
            pair = jnp.zeros((L,), jnp.float32)
            for d in range(L):
                pair = pair + plsc.load_gather(
                    stage, [rows, jnp.full((L,), d, jnp.int32)])
            bsum = jnp.zeros((L,), jnp.float32)
            brow = rows * N_FIELDS
            for f in range(N_FIELDS):
                bsum = bsum + plsc.load_gather(biasv.at[b], [brow + f])
            z = 0.5 * pair + bsum + w0vec
            out16 = 5.5 / (1.0 + jnp.exp(-z))
            off = pl.multiple_of(c * CHUNK + g * L, L)
            logits[pl.ds(off, L)] = out16

    out_base = pl.multiple_of(wid * S_PER_W, S_PER_W)
    pltpu.sync_copy(logits, out_hbm.at[pl.ds(out_base, S_PER_W)])


_fm_call = pl.kernel(
    _fm_body,
    out_type=jax.ShapeDtypeStruct((BATCH,), jnp.float32),
    mesh=plsc.VectorSubcoreMesh(core_axis_name="c", subcore_axis_name="s"),
    scratch_types=[
        pltpu.VMEM((IDX_PER_W,), jnp.int32),
        pltpu.VMEM((2, IDX_PER_CHUNK, EMBED_DIM), jnp.float32),
        pltpu.VMEM((2, IDX_PER_CHUNK), jnp.float32),
        pltpu.VMEM((S_PER_W,), jnp.float32),
        pltpu.VMEM((CHUNK, 17), jnp.float32),
        pltpu.VMEM((L,), jnp.float32),
        pltpu.SemaphoreType.DMA,
    ],
    compiler_params=pltpu.CompilerParams(
        needs_layout_passes=False, use_tc_tiling_on_sc=False),
)


def kernel(X, w0, bias_table, emb_table):
    x_flat = X.reshape(-1).astype(jnp.int32)
    bias_flat = bias_table.reshape(-1)
    w0v = jnp.broadcast_to(w0.astype(jnp.float32), (L,))
    return _fm_call(x_flat, w0v, bias_flat, emb_table)
